# baseline (speedup 1.0000x reference)
import functools

import jax
import jax.numpy as jnp
from jax import lax
from jax.experimental import pallas as pl
from jax.experimental.pallas import tpu as pltpu
import jax.experimental.pallas.tpu_sc as plsc

N_NODES = 10000
N_EDGES = 320000
DIM = 128

NC = 2
NS = 16
L = 16

CH = 80
CHUNKS_PER_TILE = N_EDGES // (NC * NS * CH)
ROWS_PER_TILE = N_NODES // NS
ZROWS = 25
NROW = 3
NMETA = 4
UNROLL = 12


def _spmm_body(x_hbm, meta_hbm, out_hbm,
               mb0, mb1, mb2, mb3, rw0, rw1, rw2,
               zbuf, acc, sm, sg, ss, sz):
    meta = [mb0, mb1, mb2, mb3]
    rows = [rw0, rw1, rw2]
    c = lax.axis_index("c")
    s = lax.axis_index("s")
    wid = c * NS + s

    def issue_meta(q, b):
        pltpu.async_copy(meta_hbm.at[wid, q], meta[b], sm[b])

    def wait_meta(q, b):
        pltpu.make_async_copy(meta_hbm.at[wid, q], meta[b], sm[b]).wait()

    def issue_gather(bm, br):
        pltpu.async_copy(x_hbm.at[meta[bm].at[0]], rows[br], sg[br])

    def wait_gather(bm, br):
        pltpu.make_async_copy(x_hbm.at[meta[bm].at[0]], rows[br], sg[br]).wait()

    def issue_scatter(bm, br, b2):
        pltpu.async_copy(rows[br], acc.at[meta[bm].at[1]], ss[b2], add=True)

    def wait_scatter(bm, br, b2):
        pltpu.make_async_copy(rows[br], acc.at[meta[bm].at[1]], ss[b2]).wait()

    def scale(bm, br):
        rows_b = rows[br]
        meta_b = meta[bm]

        def grp_body(eg, carry):
            wv = lax.bitcast_convert_type(meta_b[2, pl.ds(eg * L, L)],
                                          jnp.float32)
            for j in range(L):
                we = wv[j]
                e = eg * L + j
                for k in range(DIM // L):
                    sl = pl.ds(k * L, L)
                    rows_b[e, sl] = rows_b[e, sl] * we
            return carry

        lax.fori_loop(0, CH // L, grp_body, 0)

    def chunk_step(q, u, *, first=False, fetch=True, gather=True,
                   gather_waits=True, q2=None, q3=None):
        b3 = (2 + u) % NROW
        b4 = (2 + u) % NMETA
        p3 = (1 + u) % NROW
        p4 = (1 + u) % NMETA
        d4 = u % NMETA
        sp = (1 + u) % 2
        wait_gather(b4, b3)
        scale(b4, b3)
        issue_scatter(b4, b3, u % 2)
        if not first:
            wait_scatter(p4, p3, sp)
        if fetch:
            issue_meta(q3 if q3 is not None else q + 3, p4)
        if gather:
            if gather_waits:
                wait_meta(q2 if q2 is not None else q + 2, d4)
            issue_gather(d4, p3)

    for b in range(3):
        pltpu.sync_copy(meta_hbm.at[wid, b], meta[b])
    issue_gather(0, 0)
    issue_gather(1, 1)

    zero16 = jnp.zeros((L,), jnp.float32)
    for i in range(ZROWS):
        for k in range(DIM // L):
            zbuf[i, pl.ds(k * L, L)] = zero16
    base_r = s * ROWS_PER_TILE
    nz = ROWS_PER_TILE // ZROWS

    def zloop(j, carry):
        pltpu.async_copy(zbuf, acc.at[pl.ds(base_r + j * ZROWS, ZROWS)], sz)
        return carry

    lax.fori_loop(0, nz, zloop, 0)

    def zdrain(j, carry):
        pltpu.make_async_copy(zbuf, acc.at[pl.ds(base_r, ZROWS)], sz).wait()
        return carry

    lax.fori_loop(0, nz, zdrain, 0)
    plsc.subcore_barrier()

    chunk_step(0, 10, first=True, gather_waits=False)
    chunk_step(1, 11)

    def pipe_body(t, carry):
        for u in range(UNROLL):
            chunk_step(2 + UNROLL * t + u, u)
        return carry

    lax.fori_loop(0, (120 // UNROLL), pipe_body, 0)

    chunk_step(122, 0, fetch=False, q2=124)
    chunk_step(123, 1, fetch=False, gather=False)
    chunk_step(124, 2, fetch=False, gather=False)
    wait_scatter((2 + 2) % NMETA, (2 + 2) % NROW, 0)

    plsc.subcore_barrier()

    pltpu.sync_copy(acc.at[pl.ds(base_r, ROWS_PER_TILE)], out_hbm.at[c, s])


_spmm_sc = functools.partial(
    pl.kernel,
    out_type=jax.ShapeDtypeStruct((NC, NS, ROWS_PER_TILE, DIM), jnp.float32),
    mesh=plsc.VectorSubcoreMesh(core_axis_name="c", subcore_axis_name="s",
                                num_cores=NC),
    scratch_types=[
        pltpu.VMEM((3, CH), jnp.int32),
        pltpu.VMEM((3, CH), jnp.int32),
        pltpu.VMEM((3, CH), jnp.int32),
        pltpu.VMEM((3, CH), jnp.int32),
        pltpu.VMEM((CH, DIM), jnp.float32),
        pltpu.VMEM((CH, DIM), jnp.float32),
        pltpu.VMEM((CH, DIM), jnp.float32),
        pltpu.VMEM((ZROWS, DIM), jnp.float32),
        pltpu.VMEM_SHARED((N_NODES, DIM), jnp.float32),
        [pltpu.SemaphoreType.DMA] * NMETA,
        [pltpu.SemaphoreType.DMA] * NROW,
        [pltpu.SemaphoreType.DMA] * 2,
        pltpu.SemaphoreType.DMA,
    ],
)(_spmm_body)


def _linear_block(parts_ref, w_ref, o_ref, *, relu):
    p = parts_ref[0] + parts_ref[1]
    y = lax.dot_general(p, w_ref[...], (((1,), (1,)), ((), ())),
                        preferred_element_type=jnp.float32)
    if relu:
        y = jnp.maximum(y, 0.0)
    o_ref[...] = y


def _tc_linear(parts, w, relu):
    bn = 2000
    return pl.pallas_call(
        functools.partial(_linear_block, relu=relu),
        grid=(N_NODES // bn,),
        in_specs=[
            pl.BlockSpec((NC, bn, DIM), lambda i: (0, i, 0)),
            pl.BlockSpec((DIM, DIM), lambda i: (0, 0)),
        ],
        out_specs=pl.BlockSpec((bn, DIM), lambda i: (i, 0)),
        out_shape=jax.ShapeDtypeStruct((N_NODES, DIM), jnp.float32),
    )(parts, w)


@jax.jit
def kernel(x, edge_index, edge_weight, W1, W2):
    shape3 = (NC * NS, CHUNKS_PER_TILE, CH)
    src = edge_index[0].astype(jnp.int32).reshape(shape3)
    dst = edge_index[1].astype(jnp.int32).reshape(shape3)
    ewb = lax.bitcast_convert_type(edge_weight, jnp.int32).reshape(shape3)
    meta = jnp.stack([src, dst, ewb], axis=2)

    p1 = _spmm_sc(x, meta).reshape(NC, N_NODES, DIM)
    h = _tc_linear(p1, W1, relu=True)
    p2 = _spmm_sc(h, meta).reshape(NC, N_NODES, DIM)
    return _tc_linear(p2, W2, relu=False)


# speedup vs baseline: 1.0633x; 1.0633x over previous
import functools

import jax
import jax.numpy as jnp
from jax import lax
from jax.experimental import pallas as pl
from jax.experimental.pallas import tpu as pltpu
import jax.experimental.pallas.tpu_sc as plsc

N_NODES = 10000
N_EDGES = 320000
DIM = 128

NC = 2
NS = 16
L = 16

CH = 80
CHUNKS_PER_TILE = N_EDGES // (NC * NS * CH)
ROWS_PER_TILE = N_NODES // NS
ZROWS = 25
NB = 3


def _spmm_body(x_hbm, sw_hbm, dst_hbm, out_hbm,
               sw0, sw1, sw2, db0, db1, db2, rw0, rw1, rw2,
               zbuf, acc, sm, sd, sg, ss, sz):
    sw = [sw0, sw1, sw2]
    db = [db0, db1, db2]
    rows = [rw0, rw1, rw2]
    c = lax.axis_index("c")
    s = lax.axis_index("s")
    wid = c * NS + s

    def issue_sw(q, b):
        pltpu.async_copy(sw_hbm.at[wid, q], sw[b], sm[b])

    def wait_sw(q, b):
        pltpu.make_async_copy(sw_hbm.at[wid, q], sw[b], sm[b]).wait()

    def issue_dst(q, b):
        pltpu.async_copy(dst_hbm.at[wid, q, 0], db[b], sd[b])

    def wait_dst(q, b):
        pltpu.make_async_copy(dst_hbm.at[wid, q, 0], db[b], sd[b]).wait()

    def issue_gather(bm, br):
        pltpu.async_copy(x_hbm.at[sw[bm].at[0]], rows[br], sg[br])

    def wait_gather(bm, br):
        pltpu.make_async_copy(x_hbm.at[sw[bm].at[0]], rows[br], sg[br]).wait()

    def issue_scatter(b):
        pltpu.async_copy(rows[b], acc.at[db[b]], ss[b], add=True)

    def wait_scatter(b):
        pltpu.make_async_copy(rows[b], acc.at[db[b]], ss[b]).wait()

    def scale(b):
        rows_b = rows[b]
        sw_b = sw[b]

        def grp_body(eg, carry):
            wv = lax.bitcast_convert_type(sw_b[2, pl.ds(eg * L, L)],
                                          jnp.float32)
            for j in range(L):
                we = wv[j]
                e = eg * L + j
                for k in range(DIM // L):
                    sl = pl.ds(k * L, L)
                    rows_b[e, sl] = rows_b[e, sl] * we
            return carry

        lax.fori_loop(0, CH // L, grp_body, 0)

    def chunk_step(q, u, *, first=False, do_next=True, do_sw=True,
                   sw_wait=True):
        b = (2 + u) % NB
        p = (1 + u) % NB
        wait_gather(b, b)
        wait_dst(q, b)
        scale(b)
        if not first:
            wait_scatter(p)
        issue_scatter(b)
        if do_next:
            issue_dst(q + 2, p)
            if sw_wait:
                wait_sw(q + 2, p)
            if do_sw:
                issue_sw(q + 3, b)
            issue_gather(p, p)

    for b in range(NB):
        pltpu.sync_copy(sw_hbm.at[wid, b], sw[b])
    issue_dst(0, 0)
    issue_dst(1, 1)
    issue_gather(0, 0)
    issue_gather(1, 1)

    zero16 = jnp.zeros((L,), jnp.float32)
    for i in range(ZROWS):
        for k in range(DIM // L):
            zbuf[i, pl.ds(k * L, L)] = zero16
    base_r = s * ROWS_PER_TILE
    nz = ROWS_PER_TILE // ZROWS

    def zloop(j, carry):
        pltpu.async_copy(zbuf, acc.at[pl.ds(base_r + j * ZROWS, ZROWS)], sz)
        return carry

    lax.fori_loop(0, nz, zloop, 0)

    def zdrain(j, carry):
        pltpu.make_async_copy(zbuf, acc.at[pl.ds(base_r, ZROWS)], sz).wait()
        return carry

    lax.fori_loop(0, nz, zdrain, 0)
    plsc.subcore_barrier()

    chunk_step(0, 1, first=True, sw_wait=False)
    chunk_step(1, 2)

    def pipe_body(t, carry):
        for u in range(NB):
            chunk_step(2 + NB * t + u, u)
        return carry

    lax.fori_loop(0, 120 // NB, pipe_body, 0)

    chunk_step(122, 0, do_sw=False)
    chunk_step(123, 1, do_next=False)
    chunk_step(124, 2, do_next=False)
    wait_scatter(1)

    plsc.subcore_barrier()

    pltpu.sync_copy(acc.at[pl.ds(base_r, ROWS_PER_TILE)], out_hbm.at[c, s])


_spmm_sc = functools.partial(
    pl.kernel,
    out_type=jax.ShapeDtypeStruct((NC, NS, ROWS_PER_TILE, DIM), jnp.float32),
    mesh=plsc.VectorSubcoreMesh(core_axis_name="c", subcore_axis_name="s",
                                num_cores=NC),
    scratch_types=[
        pltpu.VMEM((3, CH), jnp.int32),
        pltpu.VMEM((3, CH), jnp.int32),
        pltpu.VMEM((3, CH), jnp.int32),
        pltpu.VMEM((CH,), jnp.int32),
        pltpu.VMEM((CH,), jnp.int32),
        pltpu.VMEM((CH,), jnp.int32),
        pltpu.VMEM((CH, DIM), jnp.float32),
        pltpu.VMEM((CH, DIM), jnp.float32),
        pltpu.VMEM((CH, DIM), jnp.float32),
        pltpu.VMEM((ZROWS, DIM), jnp.float32),
        pltpu.VMEM_SHARED((N_NODES, DIM), jnp.float32),
        [pltpu.SemaphoreType.DMA] * NB,
        [pltpu.SemaphoreType.DMA] * NB,
        [pltpu.SemaphoreType.DMA] * NB,
        [pltpu.SemaphoreType.DMA] * NB,
        pltpu.SemaphoreType.DMA,
    ],
)(_spmm_body)


def _linear_block(parts_ref, w_ref, o_ref, *, relu):
    p = parts_ref[0] + parts_ref[1]
    y = lax.dot_general(p, w_ref[...], (((1,), (1,)), ((), ())),
                        preferred_element_type=jnp.float32)
    if relu:
        y = jnp.maximum(y, 0.0)
    o_ref[...] = y


def _tc_linear(parts, w, relu):
    bn = 2000
    return pl.pallas_call(
        functools.partial(_linear_block, relu=relu),
        grid=(N_NODES // bn,),
        in_specs=[
            pl.BlockSpec((NC, bn, DIM), lambda i: (0, i, 0)),
            pl.BlockSpec((DIM, DIM), lambda i: (0, 0)),
        ],
        out_specs=pl.BlockSpec((bn, DIM), lambda i: (i, 0)),
        out_shape=jax.ShapeDtypeStruct((N_NODES, DIM), jnp.float32),
    )(parts, w)


@jax.jit
def kernel(x, edge_index, edge_weight, W1, W2):
    shape3 = (NC * NS, CHUNKS_PER_TILE, CH)
    src = edge_index[0].astype(jnp.int32).reshape(shape3)
    dst = edge_index[1].astype(jnp.int32).reshape(shape3)
    ewb = lax.bitcast_convert_type(edge_weight, jnp.int32).reshape(shape3)
    sw_meta = jnp.stack([src, dst, ewb], axis=2)

    dst4 = dst.reshape(NC * NS, CHUNKS_PER_TILE, 1, CH)
    p1 = _spmm_sc(x, sw_meta, dst4).reshape(NC, N_NODES, DIM)
    h = _tc_linear(p1, W1, relu=True)
    p2 = _spmm_sc(h, sw_meta, dst4).reshape(NC, N_NODES, DIM)
    return _tc_linear(p2, W2, relu=False)


# speedup vs baseline: 1.1514x; 1.0828x over previous
import functools

import jax
import jax.numpy as jnp
from jax import lax
from jax.experimental import pallas as pl
from jax.experimental.pallas import tpu as pltpu
import jax.experimental.pallas.tpu_sc as plsc

N_NODES = 10000
N_EDGES = 320000
DIM = 128

NC = 2
NS = 16
L = 16

CH = 80
CHUNKS_PER_TILE = N_EDGES // (NC * NS * CH)
ROWS_PER_TILE = N_NODES // NS
ZROWS = 25
NBUF = 3


def _spmm_body(x_hbm, src_hbm, dst_hbm, w_hbm, out_hbm,
               sb0, sb1, sb2, db0, db1, db2,
               wb0, wb1, wb2, rw0, rw1, rw2,
               zbuf, acc, ssrc, sg, sw, sd, ss, sz):
    sb = [sb0, sb1, sb2]
    db = [db0, db1, db2]
    wb = [wb0, wb1, wb2]
    rows = [rw0, rw1, rw2]
    c = lax.axis_index("c")
    s = lax.axis_index("s")
    wid = c * NS + s

    def issue_src(q, b):
        pltpu.async_copy(src_hbm.at[wid, q], sb[b], ssrc[b])

    def wait_src(q, b):
        pltpu.make_async_copy(src_hbm.at[wid, q], sb[b], ssrc[b]).wait()

    def issue_dw(q, b):
        pltpu.async_copy(dst_hbm.at[wid, q], db[b], sd[b])
        pltpu.async_copy(w_hbm.at[wid, q], wb[b], sw[b])

    def wait_dw(q, b):
        pltpu.make_async_copy(dst_hbm.at[wid, q], db[b], sd[b]).wait()
        pltpu.make_async_copy(w_hbm.at[wid, q], wb[b], sw[b]).wait()

    def issue_gather(b):
        pltpu.async_copy(x_hbm.at[sb[b]], rows[b], sg[b])

    def wait_gather(b):
        pltpu.make_async_copy(x_hbm.at[sb[b]], rows[b], sg[b]).wait()

    def issue_scatter(b):
        pltpu.async_copy(rows[b], acc.at[db[b]], ss[b], add=True)

    def wait_scatter(b):
        pltpu.make_async_copy(rows[b], acc.at[db[b]], ss[b]).wait()

    def scale(b):
        rows_b = rows[b]
        wb_b = wb[b]

        def grp_body(eg, carry):
            wv = wb_b[pl.ds(eg * L, L)]
            for j in range(L):
                we = wv[j]
                e = eg * L + j
                for k in range(DIM // L):
                    sl = pl.ds(k * L, L)
                    rows_b[e, sl] = rows_b[e, sl] * we
            return carry

        lax.fori_loop(0, CH // L, grp_body, 0)

    for b in range(NBUF):
        pltpu.sync_copy(src_hbm.at[wid, b], sb[b])
    issue_dw(0, 0)
    issue_dw(1, 1)
    issue_gather(0)
    issue_gather(1)

    zero16 = jnp.zeros((L,), jnp.float32)
    for i in range(ZROWS):
        for k in range(DIM // L):
            zbuf[i, pl.ds(k * L, L)] = zero16
    base_r = s * ROWS_PER_TILE
    nz = ROWS_PER_TILE // ZROWS

    def zloop(j, carry):
        pltpu.async_copy(zbuf, acc.at[pl.ds(base_r + j * ZROWS, ZROWS)], sz)
        return carry

    lax.fori_loop(0, nz, zloop, 0)

    def zdrain(j, carry):
        pltpu.make_async_copy(zbuf, acc.at[pl.ds(base_r, ZROWS)], sz).wait()
        return carry

    lax.fori_loop(0, nz, zdrain, 0)
    plsc.subcore_barrier()

    wait_gather(0)
    wait_dw(0, 0)
    scale(0)
    issue_scatter(0)
    issue_dw(2, 2)
    issue_gather(2)
    issue_src(3, 0)
    wait_gather(1)
    wait_dw(1, 1)
    scale(1)
    wait_scatter(0)
    issue_scatter(1)
    wait_src(3, 0)
    issue_dw(3, 0)
    issue_gather(0)
    issue_src(4, 1)

    n_steps = (CHUNKS_PER_TILE - 2) // NBUF

    def pipe_body(t, carry):
        for j in range(NBUF):
            q = 2 + NBUF * t + j
            b = (2 + j) % NBUF
            bp = (j + 1) % NBUF
            wait_gather(b)
            wait_dw(q, b)
            scale(b)
            wait_scatter(bp)
            issue_scatter(b)
            if j == 0:
                wait_src(q + 2, bp)
                issue_dw(q + 2, bp)
                issue_gather(bp)
            else:
                @pl.when(t < n_steps - 1)
                def _():
                    wait_src(q + 2, bp)
                    issue_dw(q + 2, bp)
                    issue_gather(bp)

            @pl.when(t < n_steps - 1)
            def _():
                issue_src(q + 3, b)
        return carry

    lax.fori_loop(0, n_steps, pipe_body, 0)

    wait_scatter(1)

    plsc.subcore_barrier()

    pltpu.sync_copy(acc.at[pl.ds(base_r, ROWS_PER_TILE)], out_hbm.at[c, s])


_spmm_sc = functools.partial(
    pl.kernel,
    out_type=jax.ShapeDtypeStruct((NC, NS, ROWS_PER_TILE, DIM), jnp.float32),
    mesh=plsc.VectorSubcoreMesh(core_axis_name="c", subcore_axis_name="s",
                                num_cores=NC),
    scratch_types=[
        pltpu.VMEM((CH,), jnp.int32),
        pltpu.VMEM((CH,), jnp.int32),
        pltpu.VMEM((CH,), jnp.int32),
        pltpu.VMEM((CH,), jnp.int32),
        pltpu.VMEM((CH,), jnp.int32),
        pltpu.VMEM((CH,), jnp.int32),
        pltpu.VMEM((CH,), jnp.float32),
        pltpu.VMEM((CH,), jnp.float32),
        pltpu.VMEM((CH,), jnp.float32),
        pltpu.VMEM((CH, DIM), jnp.float32),
        pltpu.VMEM((CH, DIM), jnp.float32),
        pltpu.VMEM((CH, DIM), jnp.float32),
        pltpu.VMEM((ZROWS, DIM), jnp.float32),
        pltpu.VMEM_SHARED((N_NODES, DIM), jnp.float32),
        [pltpu.SemaphoreType.DMA] * NBUF,
        [pltpu.SemaphoreType.DMA] * NBUF,
        [pltpu.SemaphoreType.DMA] * NBUF,
        [pltpu.SemaphoreType.DMA] * NBUF,
        [pltpu.SemaphoreType.DMA] * NBUF,
        pltpu.SemaphoreType.DMA,
    ],
)(_spmm_body)


def _linear_block(parts_ref, w_ref, o_ref, *, relu):
    p = parts_ref[0] + parts_ref[1]
    y = lax.dot_general(p, w_ref[...], (((1,), (1,)), ((), ())),
                        preferred_element_type=jnp.float32)
    if relu:
        y = jnp.maximum(y, 0.0)
    o_ref[...] = y


def _tc_linear(parts, w, relu):
    bn = 2000
    return pl.pallas_call(
        functools.partial(_linear_block, relu=relu),
        grid=(N_NODES // bn,),
        in_specs=[
            pl.BlockSpec((NC, bn, DIM), lambda i: (0, i, 0)),
            pl.BlockSpec((DIM, DIM), lambda i: (0, 0)),
        ],
        out_specs=pl.BlockSpec((bn, DIM), lambda i: (i, 0)),
        out_shape=jax.ShapeDtypeStruct((N_NODES, DIM), jnp.float32),
    )(parts, w)


@jax.jit
def kernel(x, edge_index, edge_weight, W1, W2):
    shape3 = (NC * NS, CHUNKS_PER_TILE, CH)
    src = edge_index[0].astype(jnp.int32).reshape(shape3)
    dst = edge_index[1].astype(jnp.int32).reshape(shape3)
    ew = edge_weight.reshape(shape3)

    p1 = _spmm_sc(x, src, dst, ew).reshape(NC, N_NODES, DIM)
    h = _tc_linear(p1, W1, relu=True)
    p2 = _spmm_sc(h, src, dst, ew).reshape(NC, N_NODES, DIM)
    return _tc_linear(p2, W2, relu=False)
